# Initial kernel scaffold; baseline (speedup 1.0000x reference)
import jax
import jax.numpy as jnp
from jax.experimental import pallas as pl


def kernel(con_features, var_features, obj_features, cv_edge_indices, cv_edge_features, ov_edge_indices, ov_edge_features, num_graphs, var_batch, con_batch, obj_batch, params):
    raise NotImplementedError("write your pallas kernel here")


# speedup vs baseline: 1.4527x; 1.4527x over previous
import functools

import jax
import jax.numpy as jnp
from jax import lax
from jax.experimental import pallas as pl
from jax.experimental.pallas import tpu as pltpu
from jax.experimental.pallas import tpu_sc as plsc

EMB = 128
NG = 16
NTILES = 16
NCORES = 2
RNG = 12288
ROWS_PER_TILE = RNG // NTILES
CHA = 2048
GCH = 128
MBUF = CHA + GCH + 16

_F32 = jnp.float32
_I32 = jnp.int32


def _cdiv(a, b):
    return (a + b - 1) // b


_GDN = lax.GatherDimensionNumbers(
    offset_dims=(), collapsed_slice_dims=(0,), start_index_map=(0,))


def _take16(x, idx):
    return lax.gather(x, idx[:, None], _GDN, (1,),
                      mode=lax.GatherScatterMode.PROMISE_IN_BOUNDS)


def _prefix16(v):
    io = lax.iota(_I32, 16)
    pos = v
    for sh in (1, 2, 4, 8):
        idxc = jnp.maximum(io - sh, 0)
        pos = pos + jnp.where(io >= sh, _take16(pos, idxc), 0)
    return pos


def _sc_agg_body(src_hbm, s_hbm, t_hbm, c_hbm, w_hbm, b_hbm, out_hbm,
                 wv, bv, tbuf, sbuf, cbuf, mt, ms, mc, rows, g_ms, g_mt,
                 acc, sem, *, S, R, n_tgt_pad):
    cid = lax.axis_index("c")
    sid = lax.axis_index("s")
    pltpu.sync_copy(w_hbm, wv)
    pltpu.sync_copy(b_hbm, bv)
    base = sid * S

    chunks = []
    off = 0
    while off < S:
        sz = min(CHA, S - off)
        chunks.append((off, sz))
        off += sz

    zero16 = jnp.zeros((16,), _F32)
    trash_t = jnp.full((16,), RNG, _I32)
    zero16i = jnp.zeros((16,), _I32)

    def range_body(r, _carry):
        core_owns = (r % NCORES) == cid
        lo = r * RNG

        @pl.when(core_owns)
        def _zero():
            def zb(j, _):
                rows[j // 8, pl.ds((j % 8) * 16, 16)] = zero16
                return 0
            lax.fori_loop(0, 128 * 8, zb, 0)
            for j in range(ROWS_PER_TILE // GCH):
                pltpu.sync_copy(rows, acc.at[pl.ds(sid * ROWS_PER_TILE + j * GCH, GCH)])

            @pl.when(sid == 0)
            def _():
                pltpu.sync_copy(rows.at[pl.ds(0, 8)], acc.at[pl.ds(RNG, 8)])

        plsc.subcore_barrier()

        def drain(ch, _):
            for j in range(GCH // 16):
                g_ms[pl.ds(j * 16, 16)] = ms[pl.ds(ch * GCH + j * 16, 16)]
                g_mt[pl.ds(j * 16, 16)] = mt[pl.ds(ch * GCH + j * 16, 16)]
            pltpu.async_copy(src_hbm.at[g_ms], rows, sem).wait()

            def eb(e, _):
                cs = mc[pl.ds(ch * GCH + e, 16)][0]
                for g in range(8):
                    sl = pl.ds(g * 16, 16)
                    rows[e, sl] = jnp.maximum(rows[e, sl] + cs * wv[sl] + bv[sl], 0.0)
                return 0
            lax.fori_loop(0, GCH, eb, 0)
            pltpu.sync_copy(rows, acc.at[g_mt], add=True)
            return 0

        @pl.when(core_owns)
        def _accum():
            cnt = jnp.int32(0)
            for (coff, csz) in chunks:
                pltpu.sync_copy(t_hbm.at[pl.ds(base + coff, csz)], tbuf.at[pl.ds(0, csz)])
                pltpu.sync_copy(s_hbm.at[pl.ds(base + coff, csz)], sbuf.at[pl.ds(0, csz)])
                pltpu.sync_copy(c_hbm.at[pl.ds(base + coff, csz)], cbuf.at[pl.ds(0, csz)])

                def cpb(g, cnt):
                    tv = tbuf[pl.ds(g * 16, 16)]
                    sv = sbuf[pl.ds(g * 16, 16)]
                    cv = cbuf[pl.ds(g * 16, 16)]
                    m = (tv >= lo) & (tv < lo + RNG)
                    io = lax.iota(_I32, 16)
                    pos = _prefix16(jnp.where(m, 1, 0))
                    sh = jnp.where(m, io - pos + 1, 0)
                    vt = tv - lo
                    vs = sv
                    vc = cv
                    for b in (1, 2, 4, 8):
                        idxb = jnp.minimum(io + b, 15)
                        shv = _take16(sh, idxb)
                        claim = (shv & b) != 0
                        depart = (sh & b) != 0
                        vt = jnp.where(claim, _take16(vt, idxb), vt)
                        vs = jnp.where(claim, _take16(vs, idxb), vs)
                        vc = jnp.where(claim, _take16(vc, idxb), vc)
                        sh = jnp.where(claim, shv & ~b, jnp.where(depart, 0, sh))
                    mt[pl.ds(cnt, 16)] = vt
                    ms[pl.ds(cnt, 16)] = vs
                    mc[pl.ds(cnt, 16)] = vc
                    return cnt + pos[15]

                cnt = lax.fori_loop(0, csz // 16, cpb, cnt)

                full = cnt // GCH
                lax.fori_loop(0, full, drain, 0)

                @pl.when(full > 0)
                def _move_tail():
                    for j in range(GCH // 16):
                        v1 = mt[pl.ds(full * GCH + j * 16, 16)]
                        mt[pl.ds(j * 16, 16)] = v1
                        v2 = ms[pl.ds(full * GCH + j * 16, 16)]
                        ms[pl.ds(j * 16, 16)] = v2
                        v3 = mc[pl.ds(full * GCH + j * 16, 16)]
                        mc[pl.ds(j * 16, 16)] = v3

                cnt = cnt - full * GCH

            for g in range(GCH // 16):
                mt[pl.ds(cnt + g * 16, 16)] = trash_t
                ms[pl.ds(cnt + g * 16, 16)] = zero16i
                mc[pl.ds(cnt + g * 16, 16)] = zero16

            nch = (cnt + (GCH - 1)) // GCH
            lax.fori_loop(0, nch, drain, 0)

        plsc.subcore_barrier()

        @pl.when(core_owns)
        def _writeout():
            pltpu.sync_copy(acc.at[pl.ds(sid * ROWS_PER_TILE, ROWS_PER_TILE)],
                            out_hbm.at[pl.ds(lo + sid * ROWS_PER_TILE, ROWS_PER_TILE)])

        plsc.subcore_barrier()
        return 0

    lax.fori_loop(0, R, range_body, 0)


@functools.partial(jax.jit, static_argnames=("n_tgt",))
def _sc_aggregate(src, s_idx, t_idx, coef, w, b, n_tgt):
    E = int(s_idx.shape[0])
    S = _cdiv(E, NTILES * 16) * 16
    EP = S * NTILES
    R = _cdiv(n_tgt, RNG)
    n_tgt_pad = R * RNG

    s_p = jnp.pad(s_idx, (0, EP - E))
    t_p = jnp.pad(t_idx, (0, EP - E), constant_values=-1)
    c_p = jnp.pad(coef, (0, EP - E))

    mesh = plsc.VectorSubcoreMesh(core_axis_name="c", subcore_axis_name="s")
    MCAP = MBUF
    body = functools.partial(_sc_agg_body, S=S, R=R, n_tgt_pad=n_tgt_pad)
    out = pl.kernel(
        body,
        out_type=jax.ShapeDtypeStruct((n_tgt_pad + 8, EMB), _F32),
        mesh=mesh,
        scratch_types=[
            pltpu.VMEM((EMB,), _F32),
            pltpu.VMEM((EMB,), _F32),
            pltpu.VMEM((CHA,), _I32),
            pltpu.VMEM((CHA,), _I32),
            pltpu.VMEM((CHA,), _F32),
            pltpu.VMEM((MCAP,), _I32),
            pltpu.VMEM((MCAP,), _I32),
            pltpu.VMEM((MCAP,), _F32),
            pltpu.VMEM((GCH, EMB), _F32),
            pltpu.VMEM((GCH,), _I32),
            pltpu.VMEM((GCH,), _I32),
            pltpu.VMEM_SHARED((RNG + 8, EMB), _F32),
            pltpu.SemaphoreType.DMA,
        ],
    )(src, s_p, t_p, c_p, w, b)
    return out[:n_tgt]


_BLK = 2048


def _mlp2_body(x_ref, w1_ref, b1_ref, w2_ref, b2_ref, o_ref):
    x = x_ref[...]
    h = jnp.maximum(jnp.dot(x, w1_ref[...], preferred_element_type=_F32) + b1_ref[...], 0.0)
    o_ref[...] = jnp.maximum(jnp.dot(h, w2_ref[...], preferred_element_type=_F32) + b2_ref[...], 0.0)


def _mlp2(x, w1, b1, w2, b2):
    n, din = x.shape
    grid = _cdiv(n, _BLK)
    return pl.pallas_call(
        _mlp2_body,
        grid=(grid,),
        in_specs=[
            pl.BlockSpec((_BLK, din), lambda i: (i, 0)),
            pl.BlockSpec((din, EMB), lambda i: (0, 0)),
            pl.BlockSpec((1, EMB), lambda i: (0, 0)),
            pl.BlockSpec((EMB, EMB), lambda i: (0, 0)),
            pl.BlockSpec((1, EMB), lambda i: (0, 0)),
        ],
        out_specs=pl.BlockSpec((_BLK, EMB), lambda i: (i, 0)),
        out_shape=jax.ShapeDtypeStruct((n, EMB), _F32),
    )(x, w1, b1.reshape(1, EMB), w2, b2.reshape(1, EMB))


def _gin_update_body(t_ref, a_ref, eps_ref, w1_ref, b1_ref, w2_ref, b2_ref, o_ref):
    h = (1.0 + eps_ref[0, 0]) * t_ref[...] + a_ref[...]
    h = jnp.maximum(jnp.dot(h, w1_ref[...], preferred_element_type=_F32) + b1_ref[...], 0.0)
    o_ref[...] = jnp.dot(h, w2_ref[...], preferred_element_type=_F32) + b2_ref[...]


def _gin_update(tgt, agg, eps, w1, b1, w2, b2):
    n = tgt.shape[0]
    grid = _cdiv(n, _BLK)
    return pl.pallas_call(
        _gin_update_body,
        grid=(grid,),
        in_specs=[
            pl.BlockSpec((_BLK, EMB), lambda i: (i, 0)),
            pl.BlockSpec((_BLK, EMB), lambda i: (i, 0)),
            pl.BlockSpec((1, 1), lambda i: (0, 0)),
            pl.BlockSpec((EMB, EMB), lambda i: (0, 0)),
            pl.BlockSpec((1, EMB), lambda i: (0, 0)),
            pl.BlockSpec((EMB, EMB), lambda i: (0, 0)),
            pl.BlockSpec((1, EMB), lambda i: (0, 0)),
        ],
        out_specs=pl.BlockSpec((_BLK, EMB), lambda i: (i, 0)),
        out_shape=jax.ShapeDtypeStruct((n, EMB), _F32),
    )(tgt, agg, eps.reshape(1, 1), w1, b1.reshape(1, EMB), w2, b2.reshape(1, EMB))


def _gatt1_body(x_ref, bt_ref, wa_ref, ba_ref, wb_ref, bb_ref,
                g2_ref, gmax_ref, *, n):
    i = pl.program_id(0)
    x = x_ref[...]
    g = jnp.maximum(jnp.dot(x, wa_ref[...], preferred_element_type=_F32) + ba_ref[...], 0.0)
    g2 = jnp.maximum(jnp.dot(g, wb_ref[...], preferred_element_type=_F32) + bb_ref[...], 0.0)
    g2_ref[...] = g2
    rows = lax.broadcasted_iota(_I32, (_BLK, 1), 0) + i * _BLK
    valid = rows < n
    onehot = (bt_ref[...] == lax.broadcasted_iota(_I32, (1, NG), 1)) & valid
    gm = jnp.max(jnp.where(onehot, g2, -jnp.inf), axis=0, keepdims=True)
    prev = jnp.where(i == 0, jnp.full((1, NG), -jnp.inf), gmax_ref[...])
    gmax_ref[...] = jnp.maximum(prev, gm)


def _gatt2_body(x_ref, g2_ref, bt_ref, gmax_ref, num_ref, den_ref, *, n):
    i = pl.program_id(0)
    x = x_ref[...]
    g2 = g2_ref[...]
    rows = lax.broadcasted_iota(_I32, (_BLK, 1), 0) + i * _BLK
    valid = rows < n
    onehot_b = (bt_ref[...] == lax.broadcasted_iota(_I32, (1, NG), 1)) & valid
    onehot = onehot_b.astype(_F32)
    gmax_row = jnp.sum(onehot * gmax_ref[...], axis=1, keepdims=True)
    ex = jnp.where(valid, jnp.exp(g2 - gmax_row), 0.0)
    woh = onehot * ex
    den_part = lax.dot_general(woh, ex * 0.0 + 1.0, (((0,), (0,)), ((), ())),
                               preferred_element_type=_F32)
    num_part = lax.dot_general(woh, x, (((0,), (0,)), ((), ())),
                               preferred_element_type=_F32)
    pn = jnp.where(i == 0, jnp.zeros((NG, EMB), _F32), num_ref[...])
    pd = jnp.where(i == 0, jnp.zeros((NG, 1), _F32), den_ref[...])
    num_ref[...] = pn + num_part
    den_ref[...] = pd + den_part


def _gatt(x, batch2d, wa, ba, wb, bb):
    n = x.shape[0]
    grid = _cdiv(n, _BLK)
    g2, gmax = pl.pallas_call(
        functools.partial(_gatt1_body, n=n),
        grid=(grid,),
        in_specs=[
            pl.BlockSpec((_BLK, EMB), lambda i: (i, 0)),
            pl.BlockSpec((_BLK, 1), lambda i: (i, 0)),
            pl.BlockSpec((EMB, EMB), lambda i: (0, 0)),
            pl.BlockSpec((1, EMB), lambda i: (0, 0)),
            pl.BlockSpec((EMB, 1), lambda i: (0, 0)),
            pl.BlockSpec((1, 1), lambda i: (0, 0)),
        ],
        out_specs=[
            pl.BlockSpec((_BLK, 1), lambda i: (i, 0)),
            pl.BlockSpec((1, NG), lambda i: (0, 0)),
        ],
        out_shape=[
            jax.ShapeDtypeStruct((n, 1), _F32),
            jax.ShapeDtypeStruct((1, NG), _F32),
        ],
    )(x, batch2d, wa, ba.reshape(1, EMB), wb, bb.reshape(1, 1))
    num, den = pl.pallas_call(
        functools.partial(_gatt2_body, n=n),
        grid=(grid,),
        in_specs=[
            pl.BlockSpec((_BLK, EMB), lambda i: (i, 0)),
            pl.BlockSpec((_BLK, 1), lambda i: (i, 0)),
            pl.BlockSpec((_BLK, 1), lambda i: (i, 0)),
            pl.BlockSpec((1, NG), lambda i: (0, 0)),
        ],
        out_specs=[
            pl.BlockSpec((NG, EMB), lambda i: (0, 0)),
            pl.BlockSpec((NG, 1), lambda i: (0, 0)),
        ],
        out_shape=[
            jax.ShapeDtypeStruct((NG, EMB), _F32),
            jax.ShapeDtypeStruct((NG, 1), _F32),
        ],
    )(x, g2, batch2d, gmax)
    return num, den


def _head_body(ns_ref, ds_ref, nn_ref, dn_ref, nr_ref, dr_ref,
               w1_ref, b1_ref, w2_ref, b2_ref, o_ref):
    st = ns_ref[...] / (ds_ref[...] + 1e-16)
    nd = nn_ref[...] / (dn_ref[...] + 1e-16)
    rd = nr_ref[...] / (dr_ref[...] + 1e-16)
    x = jnp.concatenate([st, nd, rd], axis=1)
    h = jnp.maximum(jnp.dot(x, w1_ref[...], preferred_element_type=_F32) + b1_ref[...], 0.0)
    y = jnp.dot(h, w2_ref[...], preferred_element_type=_F32) + b2_ref[...]
    o_ref[...] = jax.nn.sigmoid(y)


def _head(ns, ds, nn, dn, nr, dr, w1, b1, w2, b2):
    return pl.pallas_call(
        _head_body,
        out_shape=jax.ShapeDtypeStruct((NG, 1), _F32),
    )(ns, ds, nn, dn, nr, dr, w1, b1.reshape(1, EMB), w2, b2.reshape(1, 1))


def kernel(con_features, var_features, obj_features, cv_edge_indices,
           cv_edge_features, ov_edge_indices, ov_edge_features, num_graphs,
           var_batch, con_batch, obj_batch, params):
    p = params
    n_con = con_features.shape[0]
    n_var = var_features.shape[0]
    n_obj = obj_features.shape[0]

    cv_s = cv_edge_indices[0]
    cv_t = cv_edge_indices[1]
    ov_s = ov_edge_indices[0]
    ov_t = ov_edge_indices[1]
    cv_c = cv_edge_features[:, 0]
    ov_c = ov_edge_features[:, 0]

    con = _mlp2(con_features, p['ce1_W'], p['ce1_b'], p['ce2_W'], p['ce2_b'])
    var = _mlp2(var_features, p['ve1_W'], p['ve1_b'], p['ve2_W'], p['ve2_b'])
    obj = _mlp2(obj_features, p['oe1_W'], p['oe1_b'], p['oe2_W'], p['oe2_b'])

    agg = _sc_aggregate(var, cv_t, cv_s, cv_c, p['vc_We'][0], p['vc_be'], n_con)
    con1 = _gin_update(con, agg, p['vc_eps'], p['vc_W1'], p['vc_b1'], p['vc_W2'], p['vc_b2'])
    agg = _sc_aggregate(con1, cv_s, cv_t, cv_c, p['cv_We'][0], p['cv_be'], n_var)
    var1 = _gin_update(var, agg, p['cv_eps'], p['cv_W1'], p['cv_b1'], p['cv_W2'], p['cv_b2'])

    vb = var_batch.reshape(-1, 1)
    cb = con_batch.reshape(-1, 1)
    ob = obj_batch.reshape(-1, 1)

    st_num, st_den = _gatt(jnp.concatenate([var1, con1], 0),
                           jnp.concatenate([vb, cb], 0),
                           p['g1a_W'], p['g1a_b'], p['g1b_W'], p['g1b_b'])

    agg = _sc_aggregate(var1, ov_t, ov_s, ov_c, p['vo_We'][0], p['vo_be'], n_obj)
    obj1 = _gin_update(obj, agg, p['vo_eps'], p['vo_W1'], p['vo_b1'], p['vo_W2'], p['vo_b2'])
    agg = _sc_aggregate(obj1, ov_s, ov_t, ov_c, p['ov_We'][0], p['ov_be'], n_var)
    var2 = _gin_update(var1, agg, p['ov_eps'], p['ov_W1'], p['ov_b1'], p['ov_W2'], p['ov_b2'])

    nd_num, nd_den = _gatt(jnp.concatenate([var2, obj1], 0),
                           jnp.concatenate([vb, ob], 0),
                           p['g2a_W'], p['g2a_b'], p['g2b_W'], p['g2b_b'])
    rd_num, rd_den = _gatt(jnp.concatenate([obj1, con1], 0),
                           jnp.concatenate([ob, cb], 0),
                           p['g3a_W'], p['g3a_b'], p['g3b_W'], p['g3b_b'])

    return _head(st_num, st_den, nd_num, nd_den, rd_num, rd_den,
                 p['l1_W'], p['l1_b'], p['l2_W'], p['l2_b'])


# speedup vs baseline: 1.6790x; 1.1558x over previous
import functools

import jax
import jax.numpy as jnp
from jax import lax
from jax.experimental import pallas as pl
from jax.experimental.pallas import tpu as pltpu
from jax.experimental.pallas import tpu_sc as plsc

EMB = 128
NG = 16
NTILES = 16
NCORES = 2
RNG = 13312
ROWS_PER_TILE = RNG // NTILES
CHA = 1024
GCH = 64
MBUF = CHA + 2 * GCH + 16 + 128
SSH = 17
SMASK = (1 << SSH) - 1

_F32 = jnp.float32
_I32 = jnp.int32


def _cdiv(a, b):
    return (a + b - 1) // b


_GDN = lax.GatherDimensionNumbers(
    offset_dims=(), collapsed_slice_dims=(0,), start_index_map=(0,))


def _take16(x, idx):
    return lax.gather(x, idx[:, None], _GDN, (1,),
                      mode=lax.GatherScatterMode.PROMISE_IN_BOUNDS)


def _prefix16(v):
    io = lax.iota(_I32, 16)
    pos = v
    for sh in (1, 2, 4, 8):
        idxc = jnp.maximum(io - sh, 0)
        pos = pos + jnp.where(io >= sh, _take16(pos, idxc), 0)
    return pos


def _sc_agg_body(src_hbm, s_hbm, t_hbm, c_hbm, w_hbm, b_hbm, out_hbm,
                 wv, bv, tbuf, sbuf, cbuf, mts, mc,
                 rows0, rows1, gs0, gs1, gt0, gt1,
                 acc, sem_g, sem_s, *, S, R):
    cid = lax.axis_index("c")
    sid = lax.axis_index("s")
    pltpu.sync_copy(w_hbm, wv)
    pltpu.sync_copy(b_hbm, bv)
    base = sid * S
    zero16 = jnp.zeros((16,), _F32)
    trash_p = jnp.full((16,), RNG << SSH, _I32)

    rows_b = (rows0, rows1)
    gs_b = (gs0, gs1)
    gt_b = (gt0, gt1)

    def stage_idx(ch, q):
        for j in range(GCH // 16):
            v = mts[pl.ds(ch * GCH + j * 16, 16)]
            gt_b[q][pl.ds(j * 16, 16)] = lax.shift_right_logical(v, SSH)
            gs_b[q][pl.ds(j * 16, 16)] = v & SMASK

    def range_body(r, _carry):
        core_owns = (r % NCORES) == cid
        lo = r * RNG

        @pl.when(core_owns)
        def _zero():
            def zb(j, _):
                rows0[j // 8, pl.ds((j % 8) * 16, 16)] = zero16
                return 0
            lax.fori_loop(0, GCH * 8, zb, 0)
            for j in range(ROWS_PER_TILE // GCH):
                pltpu.sync_copy(rows0, acc.at[pl.ds(sid * ROWS_PER_TILE + j * GCH, GCH)])

            @pl.when(sid == 0)
            def _():
                pltpu.sync_copy(rows0.at[pl.ds(0, 8)], acc.at[pl.ds(RNG, 8)])

        plsc.subcore_barrier()

        def handle(ch, nch, p):
            q = 1 - p
            nxt = ch + 1
            pltpu.make_async_copy(src_hbm.at[gs_b[p]], rows_b[p], sem_g).wait()

            @pl.when(nxt < nch)
            def _prefetch():
                @pl.when(ch >= 1)
                def _():
                    pltpu.make_async_copy(rows_b[q], acc.at[gt_b[q]], sem_s).wait()
                stage_idx(nxt, q)
                pltpu.async_copy(src_hbm.at[gs_b[q]], rows_b[q], sem_g)

            def eb(e, _):
                cs = mc[pl.ds(ch * GCH + e, 16)][0]
                for g in range(8):
                    sl = pl.ds(g * 16, 16)
                    rows_b[p][e, sl] = jnp.maximum(
                        rows_b[p][e, sl] + cs * wv[sl] + bv[sl], 0.0)
                return 0
            lax.fori_loop(0, GCH, eb, 0)
            pltpu.async_copy(rows_b[p], acc.at[gt_b[p]], sem_s, add=True)

        def drain(nch):
            @pl.when(nch > 0)
            def _():
                stage_idx(0, 0)
                pltpu.async_copy(src_hbm.at[gs0], rows0, sem_g)

                def pair(k, _):
                    @pl.when(2 * k < nch)
                    def _():
                        handle(2 * k, nch, 0)

                    @pl.when(2 * k + 1 < nch)
                    def _():
                        handle(2 * k + 1, nch, 1)
                    return 0
                lax.fori_loop(0, (nch + 1) // 2, pair, 0)

                @pl.when(nch >= 2)
                def _():
                    pltpu.make_async_copy(rows0, acc.at[gt0], sem_s).wait()

                @pl.when(nch >= 1)
                def _():
                    pltpu.make_async_copy(rows1, acc.at[gt1], sem_s).wait()

        @pl.when(core_owns)
        def _accum():
            def chunk_body(k, cnt):
                coff = k * CHA
                pltpu.sync_copy(t_hbm.at[pl.ds(base + coff, CHA)], tbuf)
                pltpu.sync_copy(s_hbm.at[pl.ds(base + coff, CHA)], sbuf)
                pltpu.sync_copy(c_hbm.at[pl.ds(base + coff, CHA)], cbuf)

                def cpb(g, cnt):
                    tv = tbuf[pl.ds(g * 16, 16)]
                    sv = sbuf[pl.ds(g * 16, 16)]
                    cv = cbuf[pl.ds(g * 16, 16)]
                    m = (tv >= lo) & (tv < lo + RNG)
                    io = lax.iota(_I32, 16)
                    pos = _prefix16(jnp.where(m, 1, 0))
                    sh = jnp.where(m, io - pos + 1, 0)
                    vp = lax.shift_left(tv - lo, SSH) | sv
                    vc = cv
                    for b in (1, 2, 4, 8):
                        idxb = jnp.minimum(io + b, 15)
                        shv = _take16(sh, idxb)
                        claim = (shv & b) != 0
                        depart = (sh & b) != 0
                        vp = jnp.where(claim, _take16(vp, idxb), vp)
                        vc = jnp.where(claim, _take16(vc, idxb), vc)
                        sh = jnp.where(claim, shv & ~b, jnp.where(depart, 0, sh))
                    mts[pl.ds(cnt, 16)] = vp
                    mc[pl.ds(cnt, 16)] = vc
                    return cnt + pos[15]

                cnt = lax.fori_loop(0, CHA // 16, cpb, cnt)

                full = cnt // GCH
                drain(full)

                @pl.when(full > 0)
                def _move_tail():
                    for j in range(2 * GCH // 16):
                        v1 = mts[pl.ds(full * GCH + j * 16, 16)]
                        mts[pl.ds(j * 16, 16)] = v1
                        v2 = mc[pl.ds(full * GCH + j * 16, 16)]
                        mc[pl.ds(j * 16, 16)] = v2

                return cnt - full * GCH

            cnt = lax.fori_loop(0, S // CHA, chunk_body, jnp.int32(0))

            for g in range(GCH // 16):
                mts[pl.ds(cnt + g * 16, 16)] = trash_p
                mc[pl.ds(cnt + g * 16, 16)] = zero16

            drain((cnt + (GCH - 1)) // GCH)

        plsc.subcore_barrier()

        @pl.when(core_owns)
        def _writeout():
            pltpu.sync_copy(acc.at[pl.ds(sid * ROWS_PER_TILE, ROWS_PER_TILE)],
                            out_hbm.at[pl.ds(lo + sid * ROWS_PER_TILE, ROWS_PER_TILE)])

        plsc.subcore_barrier()
        return 0

    lax.fori_loop(0, R, range_body, 0)


@functools.partial(jax.jit, static_argnames=("n_tgt",))
def _sc_aggregate(src, s_idx, t_idx, coef, w, b, n_tgt):
    E = int(s_idx.shape[0])
    S = _cdiv(E, NTILES * CHA) * CHA
    EP = S * NTILES
    R = _cdiv(n_tgt, RNG)
    n_tgt_pad = R * RNG

    s_p = jnp.pad(s_idx, (0, EP - E))
    t_p = jnp.pad(t_idx, (0, EP - E), constant_values=-1)
    c_p = jnp.pad(coef, (0, EP - E))

    mesh = plsc.VectorSubcoreMesh(core_axis_name="c", subcore_axis_name="s")
    body = functools.partial(_sc_agg_body, S=S, R=R)
    out = pl.kernel(
        body,
        out_type=jax.ShapeDtypeStruct((n_tgt_pad + 8, EMB), _F32),
        mesh=mesh,
        scratch_types=[
            pltpu.VMEM((EMB,), _F32),
            pltpu.VMEM((EMB,), _F32),
            pltpu.VMEM((CHA,), _I32),
            pltpu.VMEM((CHA,), _I32),
            pltpu.VMEM((CHA,), _F32),
            pltpu.VMEM((MBUF,), _I32),
            pltpu.VMEM((MBUF,), _F32),
            pltpu.VMEM((GCH, EMB), _F32),
            pltpu.VMEM((GCH, EMB), _F32),
            pltpu.VMEM((GCH,), _I32),
            pltpu.VMEM((GCH,), _I32),
            pltpu.VMEM((GCH,), _I32),
            pltpu.VMEM((GCH,), _I32),
            pltpu.VMEM_SHARED((RNG + 8, EMB), _F32),
            pltpu.SemaphoreType.DMA,
            pltpu.SemaphoreType.DMA,
        ],
    )(src, s_p, t_p, c_p, w, b)
    return out[:n_tgt]


_BLK = 2048


def _mlp2_body(x_ref, w1_ref, b1_ref, w2_ref, b2_ref, o_ref):
    x = x_ref[...]
    h = jnp.maximum(jnp.dot(x, w1_ref[...], preferred_element_type=_F32) + b1_ref[...], 0.0)
    o_ref[...] = jnp.maximum(jnp.dot(h, w2_ref[...], preferred_element_type=_F32) + b2_ref[...], 0.0)


def _mlp2(x, w1, b1, w2, b2):
    n, din = x.shape
    grid = _cdiv(n, _BLK)
    return pl.pallas_call(
        _mlp2_body,
        grid=(grid,),
        in_specs=[
            pl.BlockSpec((_BLK, din), lambda i: (i, 0)),
            pl.BlockSpec((din, EMB), lambda i: (0, 0)),
            pl.BlockSpec((1, EMB), lambda i: (0, 0)),
            pl.BlockSpec((EMB, EMB), lambda i: (0, 0)),
            pl.BlockSpec((1, EMB), lambda i: (0, 0)),
        ],
        out_specs=pl.BlockSpec((_BLK, EMB), lambda i: (i, 0)),
        out_shape=jax.ShapeDtypeStruct((n, EMB), _F32),
    )(x, w1, b1.reshape(1, EMB), w2, b2.reshape(1, EMB))


def _gin_update_body(t_ref, a_ref, eps_ref, w1_ref, b1_ref, w2_ref, b2_ref, o_ref):
    h = (1.0 + eps_ref[0, 0]) * t_ref[...] + a_ref[...]
    h = jnp.maximum(jnp.dot(h, w1_ref[...], preferred_element_type=_F32) + b1_ref[...], 0.0)
    o_ref[...] = jnp.dot(h, w2_ref[...], preferred_element_type=_F32) + b2_ref[...]


def _gin_update(tgt, agg, eps, w1, b1, w2, b2):
    n = tgt.shape[0]
    grid = _cdiv(n, _BLK)
    return pl.pallas_call(
        _gin_update_body,
        grid=(grid,),
        in_specs=[
            pl.BlockSpec((_BLK, EMB), lambda i: (i, 0)),
            pl.BlockSpec((_BLK, EMB), lambda i: (i, 0)),
            pl.BlockSpec((1, 1), lambda i: (0, 0)),
            pl.BlockSpec((EMB, EMB), lambda i: (0, 0)),
            pl.BlockSpec((1, EMB), lambda i: (0, 0)),
            pl.BlockSpec((EMB, EMB), lambda i: (0, 0)),
            pl.BlockSpec((1, EMB), lambda i: (0, 0)),
        ],
        out_specs=pl.BlockSpec((_BLK, EMB), lambda i: (i, 0)),
        out_shape=jax.ShapeDtypeStruct((n, EMB), _F32),
    )(tgt, agg, eps.reshape(1, 1), w1, b1.reshape(1, EMB), w2, b2.reshape(1, EMB))


def _gatt1_body(x_ref, bt_ref, wa_ref, ba_ref, wb_ref, bb_ref,
                g2_ref, gmax_ref, *, n):
    i = pl.program_id(0)
    x = x_ref[...]
    g = jnp.maximum(jnp.dot(x, wa_ref[...], preferred_element_type=_F32) + ba_ref[...], 0.0)
    g2 = jnp.maximum(jnp.dot(g, wb_ref[...], preferred_element_type=_F32) + bb_ref[...], 0.0)
    g2_ref[...] = g2
    rows = lax.broadcasted_iota(_I32, (_BLK, 1), 0) + i * _BLK
    valid = rows < n
    onehot = (bt_ref[...] == lax.broadcasted_iota(_I32, (1, NG), 1)) & valid
    gm = jnp.max(jnp.where(onehot, g2, -jnp.inf), axis=0, keepdims=True)
    prev = jnp.where(i == 0, jnp.full((1, NG), -jnp.inf), gmax_ref[...])
    gmax_ref[...] = jnp.maximum(prev, gm)


def _gatt2_body(x_ref, g2_ref, bt_ref, gmax_ref, num_ref, den_ref, *, n):
    i = pl.program_id(0)
    x = x_ref[...]
    g2 = g2_ref[...]
    rows = lax.broadcasted_iota(_I32, (_BLK, 1), 0) + i * _BLK
    valid = rows < n
    onehot_b = (bt_ref[...] == lax.broadcasted_iota(_I32, (1, NG), 1)) & valid
    onehot = onehot_b.astype(_F32)
    gmax_row = jnp.sum(onehot * gmax_ref[...], axis=1, keepdims=True)
    ex = jnp.where(valid, jnp.exp(g2 - gmax_row), 0.0)
    woh = onehot * ex
    den_part = lax.dot_general(woh, ex * 0.0 + 1.0, (((0,), (0,)), ((), ())),
                               preferred_element_type=_F32)
    num_part = lax.dot_general(woh, x, (((0,), (0,)), ((), ())),
                               preferred_element_type=_F32)
    pn = jnp.where(i == 0, jnp.zeros((NG, EMB), _F32), num_ref[...])
    pd = jnp.where(i == 0, jnp.zeros((NG, 1), _F32), den_ref[...])
    num_ref[...] = pn + num_part
    den_ref[...] = pd + den_part


def _gatt(x, batch2d, wa, ba, wb, bb):
    n = x.shape[0]
    grid = _cdiv(n, _BLK)
    g2, gmax = pl.pallas_call(
        functools.partial(_gatt1_body, n=n),
        grid=(grid,),
        in_specs=[
            pl.BlockSpec((_BLK, EMB), lambda i: (i, 0)),
            pl.BlockSpec((_BLK, 1), lambda i: (i, 0)),
            pl.BlockSpec((EMB, EMB), lambda i: (0, 0)),
            pl.BlockSpec((1, EMB), lambda i: (0, 0)),
            pl.BlockSpec((EMB, 1), lambda i: (0, 0)),
            pl.BlockSpec((1, 1), lambda i: (0, 0)),
        ],
        out_specs=[
            pl.BlockSpec((_BLK, 1), lambda i: (i, 0)),
            pl.BlockSpec((1, NG), lambda i: (0, 0)),
        ],
        out_shape=[
            jax.ShapeDtypeStruct((n, 1), _F32),
            jax.ShapeDtypeStruct((1, NG), _F32),
        ],
    )(x, batch2d, wa, ba.reshape(1, EMB), wb, bb.reshape(1, 1))
    num, den = pl.pallas_call(
        functools.partial(_gatt2_body, n=n),
        grid=(grid,),
        in_specs=[
            pl.BlockSpec((_BLK, EMB), lambda i: (i, 0)),
            pl.BlockSpec((_BLK, 1), lambda i: (i, 0)),
            pl.BlockSpec((_BLK, 1), lambda i: (i, 0)),
            pl.BlockSpec((1, NG), lambda i: (0, 0)),
        ],
        out_specs=[
            pl.BlockSpec((NG, EMB), lambda i: (0, 0)),
            pl.BlockSpec((NG, 1), lambda i: (0, 0)),
        ],
        out_shape=[
            jax.ShapeDtypeStruct((NG, EMB), _F32),
            jax.ShapeDtypeStruct((NG, 1), _F32),
        ],
    )(x, g2, batch2d, gmax)
    return num, den


def _head_body(ns_ref, ds_ref, nn_ref, dn_ref, nr_ref, dr_ref,
               w1_ref, b1_ref, w2_ref, b2_ref, o_ref):
    st = ns_ref[...] / (ds_ref[...] + 1e-16)
    nd = nn_ref[...] / (dn_ref[...] + 1e-16)
    rd = nr_ref[...] / (dr_ref[...] + 1e-16)
    x = jnp.concatenate([st, nd, rd], axis=1)
    h = jnp.maximum(jnp.dot(x, w1_ref[...], preferred_element_type=_F32) + b1_ref[...], 0.0)
    y = jnp.dot(h, w2_ref[...], preferred_element_type=_F32) + b2_ref[...]
    o_ref[...] = jax.nn.sigmoid(y)


def _head(ns, ds, nn, dn, nr, dr, w1, b1, w2, b2):
    return pl.pallas_call(
        _head_body,
        out_shape=jax.ShapeDtypeStruct((NG, 1), _F32),
    )(ns, ds, nn, dn, nr, dr, w1, b1.reshape(1, EMB), w2, b2.reshape(1, 1))


def kernel(con_features, var_features, obj_features, cv_edge_indices,
           cv_edge_features, ov_edge_indices, ov_edge_features, num_graphs,
           var_batch, con_batch, obj_batch, params):
    p = params
    n_con = con_features.shape[0]
    n_var = var_features.shape[0]
    n_obj = obj_features.shape[0]

    cv_s = cv_edge_indices[0]
    cv_t = cv_edge_indices[1]
    ov_s = ov_edge_indices[0]
    ov_t = ov_edge_indices[1]
    cv_c = cv_edge_features[:, 0]
    ov_c = ov_edge_features[:, 0]

    con = _mlp2(con_features, p['ce1_W'], p['ce1_b'], p['ce2_W'], p['ce2_b'])
    var = _mlp2(var_features, p['ve1_W'], p['ve1_b'], p['ve2_W'], p['ve2_b'])
    obj = _mlp2(obj_features, p['oe1_W'], p['oe1_b'], p['oe2_W'], p['oe2_b'])

    agg = _sc_aggregate(var, cv_t, cv_s, cv_c, p['vc_We'][0], p['vc_be'], n_con)
    con1 = _gin_update(con, agg, p['vc_eps'], p['vc_W1'], p['vc_b1'], p['vc_W2'], p['vc_b2'])
    agg = _sc_aggregate(con1, cv_s, cv_t, cv_c, p['cv_We'][0], p['cv_be'], n_var)
    var1 = _gin_update(var, agg, p['cv_eps'], p['cv_W1'], p['cv_b1'], p['cv_W2'], p['cv_b2'])

    vb = var_batch.reshape(-1, 1)
    cb = con_batch.reshape(-1, 1)
    ob = obj_batch.reshape(-1, 1)

    st_num, st_den = _gatt(jnp.concatenate([var1, con1], 0),
                           jnp.concatenate([vb, cb], 0),
                           p['g1a_W'], p['g1a_b'], p['g1b_W'], p['g1b_b'])

    agg = _sc_aggregate(var1, ov_t, ov_s, ov_c, p['vo_We'][0], p['vo_be'], n_obj)
    obj1 = _gin_update(obj, agg, p['vo_eps'], p['vo_W1'], p['vo_b1'], p['vo_W2'], p['vo_b2'])
    agg = _sc_aggregate(obj1, ov_s, ov_t, ov_c, p['ov_We'][0], p['ov_be'], n_var)
    var2 = _gin_update(var1, agg, p['ov_eps'], p['ov_W1'], p['ov_b1'], p['ov_W2'], p['ov_b2'])

    nd_num, nd_den = _gatt(jnp.concatenate([var2, obj1], 0),
                           jnp.concatenate([vb, ob], 0),
                           p['g2a_W'], p['g2a_b'], p['g2b_W'], p['g2b_b'])
    rd_num, rd_den = _gatt(jnp.concatenate([obj1, con1], 0),
                           jnp.concatenate([ob, cb], 0),
                           p['g3a_W'], p['g3a_b'], p['g3b_W'], p['g3b_b'])

    return _head(st_num, st_den, nd_num, nd_den, rd_num, rd_den,
                 p['l1_W'], p['l1_b'], p['l2_W'], p['l2_b'])
